# baseline (speedup 1.0000x reference)
import jax
import jax.numpy as jnp
from jax.experimental import pallas as pl
from jax.experimental.pallas import tpu as pltpu


def _gather_body(maps_ref, x_ref, o_ref):
    j = x_ref.shape[1]
    k = o_ref.shape[1]
    maps = maps_ref[0, :]
    cols = jax.lax.broadcasted_iota(jnp.int32, (k, j), 1)
    sel = jnp.where(cols == maps[:, None], 1.0, 0.0).astype(jnp.float32)
    o_ref[0] = jnp.dot(sel, x_ref[0], preferred_element_type=jnp.float32)


def kernel(joints, joint_maps):
    b, j, c = joints.shape
    k = joint_maps.shape[0]
    xt = jnp.transpose(joints, (2, 1, 0))
    maps = joint_maps.reshape(1, k)
    blk = 8192
    out_t = pl.pallas_call(
        _gather_body,
        grid=(c, b // blk),
        in_specs=[
            pl.BlockSpec((1, k), lambda ci, i: (0, 0)),
            pl.BlockSpec((1, j, blk), lambda ci, i: (ci, 0, i)),
        ],
        out_specs=pl.BlockSpec((1, k, blk), lambda ci, i: (ci, 0, i)),
        out_shape=jax.ShapeDtypeStruct((c, k, b), jnp.float32),
    )(maps, xt)
    return jnp.transpose(out_t, (2, 1, 0))


# speedup vs baseline: 1.1078x; 1.1078x over previous
import jax
import jax.numpy as jnp
from jax.experimental import pallas as pl
from jax.experimental.pallas import tpu as pltpu


def _gather_body(maps_ref, x_ref, o_ref):
    c, j, _ = x_ref.shape
    k = o_ref.shape[1]
    maps = maps_ref[0, :]
    cols = jax.lax.broadcasted_iota(jnp.int32, (k, j), 1)
    sel = jnp.where(cols == maps[:, None], 1.0, 0.0).astype(jnp.float32)
    for ci in range(c):
        o_ref[ci] = jnp.dot(sel, x_ref[ci], preferred_element_type=jnp.float32)


def kernel(joints, joint_maps):
    b, j, c = joints.shape
    k = joint_maps.shape[0]
    xt = jnp.transpose(joints, (2, 1, 0))
    maps = joint_maps.reshape(1, k)
    blk = 8192
    out_t = pl.pallas_call(
        _gather_body,
        grid=(b // blk,),
        in_specs=[
            pl.BlockSpec((1, k), lambda i: (0, 0)),
            pl.BlockSpec((c, j, blk), lambda i: (0, 0, i)),
        ],
        out_specs=pl.BlockSpec((c, k, blk), lambda i: (0, 0, i)),
        out_shape=jax.ShapeDtypeStruct((c, k, b), jnp.float32),
    )(maps, xt)
    return jnp.transpose(out_t, (2, 1, 0))
